# baseline (speedup 1.0000x reference)
import functools

import jax
import jax.numpy as jnp
from jax import lax
from jax.experimental import pallas as pl
from jax.experimental.pallas import tpu as pltpu
from jax.experimental.pallas import tpu_sc as plsc

N = 10000
D = 128
E = 320000
NC, NS = 2, 16
NW = NC * NS
CH = 128
NP = 10112
RPT = NP // NS
BR = 1000

NCHD = 79
EPADD = NW * NCHD * CH

NCH0, NCH1 = 85, 72
NCHM = max(NCH0, NCH1)
E0 = NS * NCH0 * CH
E1CAP = NS * NCH1 * CH

_mesh = plsc.VectorSubcoreMesh(core_axis_name="c", subcore_axis_name="s")


def _deg_body(dst3, zerosD, onesD, out, deg_sh, idx_v, ones_v, dsem):
    c = lax.axis_index("c")
    s = lax.axis_index("s")
    wid = s * NC + c
    r0 = s * RPT
    pltpu.sync_copy(zerosD.at[pl.ds(r0, RPT)], deg_sh.at[pl.ds(r0, RPT)])
    pltpu.sync_copy(onesD, ones_v)
    pltpu.sync_copy(dst3.at[wid], idx_v)
    plsc.subcore_barrier()

    K = 8

    def fire(j):
        pltpu.async_copy(ones_v, deg_sh.at[idx_v.at[j]], dsem, add=True)

    def drain():
        pltpu.make_async_copy(ones_v, deg_sh.at[idx_v.at[0]], dsem).wait()

    def prol(j, carry):
        fire(j)
        return carry

    def body(j, carry):
        fire(j + K)
        drain()
        return carry

    def epil(j, carry):
        drain()
        return carry

    lax.fori_loop(0, K, prol, 0)
    lax.fori_loop(0, NCHD - K, body, 0)
    lax.fori_loop(0, K, epil, 0)
    plsc.subcore_barrier()
    pltpu.sync_copy(deg_sh.at[pl.ds(r0, RPT)], out.at[pl.ds(c * NP + r0, RPT)])


_deg_call = pl.kernel(
    _deg_body,
    out_type=jax.ShapeDtypeStruct((NC * NP, D), jnp.float32),
    mesh=_mesh,
    scratch_types=[
        pltpu.VMEM_SHARED((NP, D), jnp.float32),
        pltpu.VMEM((NCHD, CH), jnp.int32),
        pltpu.VMEM((CH, D), jnp.float32),
        pltpu.SemaphoreType.DMA,
    ],
)


def _scat_body(table, src3, dst3, zerosD, out, acc_sh, sidx, didx, rows, gsem):
    c = lax.axis_index("c")
    s = lax.axis_index("s")
    wid = s * NC + c
    r0 = s * RPT
    pltpu.sync_copy(zerosD.at[pl.ds(r0, RPT)], acc_sh.at[pl.ds(r0, RPT)])
    pltpu.sync_copy(src3.at[wid], sidx)
    pltpu.sync_copy(dst3.at[wid], didx)
    plsc.subcore_barrier()

    nch = jnp.where(c == 0, NCH0, NCH1)

    def body(j, carry):
        pltpu.async_copy(table.at[sidx.at[j]], rows, gsem).wait()
        pltpu.sync_copy(rows, acc_sh.at[didx.at[j]], add=True)
        return carry

    lax.fori_loop(0, nch, body, 0)
    plsc.subcore_barrier()
    pltpu.sync_copy(acc_sh.at[pl.ds(r0, RPT)], out.at[pl.ds(c * NP + r0, RPT)])


_scat_call = pl.kernel(
    _scat_body,
    out_type=jax.ShapeDtypeStruct((NC * NP, D), jnp.float32),
    mesh=_mesh,
    scratch_types=[
        pltpu.VMEM_SHARED((NP, D), jnp.float32),
        pltpu.VMEM((NCHM, CH), jnp.int32),
        pltpu.VMEM((NCHM, CH), jnp.int32),
        pltpu.VMEM((CH, D), jnp.float32),
        pltpu.SemaphoreType.DMA,
    ],
)


def _pad_to(a, n, fill):
    return jnp.concatenate([a, jnp.full((n - a.shape[0],), fill, a.dtype)])


def _core_slab(flat, nch, fill):
    a = flat.reshape(NS, nch, CH)
    if nch < NCHM:
        dum = jnp.full((NS, NCHM - nch, CH), fill, flat.dtype)
        a = jnp.concatenate([a, dum], axis=1)
    return a


def _prep_edges(src, dst):
    dd = _pad_to(dst, EPADD, N).reshape(NW, NCHD, CH)
    s0 = _core_slab(src[:E0], NCH0, 0)
    d0 = _core_slab(dst[:E0], NCH0, N)
    s1 = _core_slab(_pad_to(src[E0:], E1CAP, 0), NCH1, 0)
    d1 = _core_slab(_pad_to(dst[E0:], E1CAP, N), NCH1, N)
    src3 = jnp.stack([s0, s1], axis=1).reshape(NW, NCHM, CH)
    dst3 = jnp.stack([d0, d1], axis=1).reshape(NW, NCHM, CH)
    return dd, src3, dst3


def _pre_body(x_ref, w_ref, d0_ref, d1_ref, hs_ref, dinv_ref):
    x0 = jnp.clip(x_ref[...], -100.0, 100.0)
    deg = d0_ref[...][:, 0:1] + d1_ref[...][:, 0:1] + 1.0
    dinv = lax.rsqrt(deg)
    h = jnp.dot(x0, w_ref[...], preferred_element_type=jnp.float32)
    hs_ref[...] = h * dinv
    dinv_ref[...] = jnp.broadcast_to(dinv, (BR, 16))


_pre_call = pl.pallas_call(
    _pre_body,
    grid=(N // BR,),
    in_specs=[
        pl.BlockSpec((BR, D), lambda i: (i, 0)),
        pl.BlockSpec((D, D), lambda i: (0, 0)),
        pl.BlockSpec((BR, D), lambda i: (i, 0)),
        pl.BlockSpec((BR, D), lambda i: (i, 0)),
    ],
    out_specs=[
        pl.BlockSpec((BR, D), lambda i: (i, 0)),
        pl.BlockSpec((BR, 16), lambda i: (i, 0)),
    ],
    out_shape=[
        jax.ShapeDtypeStruct((N, D), jnp.float32),
        jax.ShapeDtypeStruct((N, 16), jnp.float32),
    ],
)


def _mid_body(p0_ref, p1_ref, hs_ref, dinv_ref, b_ref, w_ref, out_ref):
    dinv = dinv_ref[...][:, 0:1]
    y = dinv * (p0_ref[...] + p1_ref[...] + hs_ref[...]) + b_ref[...]
    y = jnp.maximum(y, 0.0)
    out_ref[...] = jnp.dot(y, w_ref[...], preferred_element_type=jnp.float32) * dinv


_mid_call = pl.pallas_call(
    _mid_body,
    grid=(N // BR,),
    in_specs=[
        pl.BlockSpec((BR, D), lambda i: (i, 0)),
        pl.BlockSpec((BR, D), lambda i: (i, 0)),
        pl.BlockSpec((BR, D), lambda i: (i, 0)),
        pl.BlockSpec((BR, 16), lambda i: (i, 0)),
        pl.BlockSpec((1, D), lambda i: (0, 0)),
        pl.BlockSpec((D, D), lambda i: (0, 0)),
    ],
    out_specs=pl.BlockSpec((BR, D), lambda i: (i, 0)),
    out_shape=jax.ShapeDtypeStruct((N, D), jnp.float32),
)


def _fin_body(q0_ref, q1_ref, hs_ref, dinv_ref, b_ref, x_ref, wh_ref, wx_ref,
              bg_ref, out_ref):
    x0 = jnp.clip(x_ref[...], -100.0, 100.0)
    dinv = dinv_ref[...][:, 0:1]
    h2 = dinv * (q0_ref[...] + q1_ref[...] + hs_ref[...]) + b_ref[...]
    h = jnp.maximum(h2, 0.0) + x0
    g = jax.nn.sigmoid(
        jnp.dot(h, wh_ref[...], preferred_element_type=jnp.float32)
        + jnp.dot(x0, wx_ref[...], preferred_element_type=jnp.float32)
        + bg_ref[...]
    )
    out_ref[...] = g * h + (1.0 - g) * x0


_fin_call = pl.pallas_call(
    _fin_body,
    grid=(N // BR,),
    in_specs=[
        pl.BlockSpec((BR, D), lambda i: (i, 0)),
        pl.BlockSpec((BR, D), lambda i: (i, 0)),
        pl.BlockSpec((BR, D), lambda i: (i, 0)),
        pl.BlockSpec((BR, 16), lambda i: (i, 0)),
        pl.BlockSpec((1, D), lambda i: (0, 0)),
        pl.BlockSpec((BR, D), lambda i: (i, 0)),
        pl.BlockSpec((D, D), lambda i: (0, 0)),
        pl.BlockSpec((D, D), lambda i: (0, 0)),
        pl.BlockSpec((1, D), lambda i: (0, 0)),
    ],
    out_specs=pl.BlockSpec((BR, D), lambda i: (i, 0)),
    out_shape=jax.ShapeDtypeStruct((N, D), jnp.float32),
)


@jax.jit
def kernel(x, edge_index, W1, b1, W2, b2, Wg, bg):
    src = edge_index[0].astype(jnp.int32)
    dst = edge_index[1].astype(jnp.int32)
    dd, src3, dst3 = _prep_edges(src, dst)
    zerosD = jnp.zeros((NP, D), jnp.float32)
    onesD = jnp.ones((CH, D), jnp.float32)

    degp = _deg_call(dd, zerosD, onesD)
    d0, d1 = degp[0:N], degp[NP:NP + N]

    hs1, dinv16 = _pre_call(x, W1, d0, d1)

    acc1 = _scat_call(hs1, src3, dst3, zerosD)
    hs2 = _mid_call(acc1[0:N], acc1[NP:NP + N], hs1, dinv16,
                    b1.reshape(1, D), W2)

    acc2 = _scat_call(hs2, src3, dst3, zerosD)
    out = _fin_call(acc2[0:N], acc2[NP:NP + N], hs2, dinv16,
                    b2.reshape(1, D), x, Wg[:D], Wg[D:], bg.reshape(1, D))
    return out


# speedup vs baseline: 1.0513x; 1.0513x over previous
import functools

import jax
import jax.numpy as jnp
from jax import lax
from jax.experimental import pallas as pl
from jax.experimental.pallas import tpu as pltpu
from jax.experimental.pallas import tpu_sc as plsc

N = 10000
D = 128
E = 320000
NC, NS = 2, 16
NW = NC * NS
CH = 128
NP = 10112
RPT = NP // NS
BR = 1000

NCHD = 79
EPADD = NW * NCHD * CH

NCH0, NCH1 = 92, 65
NCHM = max(NCH0, NCH1)
E0 = NS * NCH0 * CH
E1CAP = NS * NCH1 * CH

_mesh = plsc.VectorSubcoreMesh(core_axis_name="c", subcore_axis_name="s")


def _deg_body(dst3, zerosD, onesD, out, deg_sh, idx_v, ones_v, dsem):
    c = lax.axis_index("c")
    s = lax.axis_index("s")
    wid = s * NC + c
    r0 = s * RPT
    pltpu.sync_copy(zerosD.at[pl.ds(r0, RPT)], deg_sh.at[pl.ds(r0, RPT)])
    pltpu.sync_copy(onesD, ones_v)
    pltpu.sync_copy(dst3.at[wid], idx_v)
    plsc.subcore_barrier()

    K = 8

    def fire(j):
        pltpu.async_copy(ones_v, deg_sh.at[idx_v.at[j]], dsem, add=True)

    def drain():
        pltpu.make_async_copy(ones_v, deg_sh.at[idx_v.at[0]], dsem).wait()

    def prol(j, carry):
        fire(j)
        return carry

    def body(j, carry):
        fire(j + K)
        drain()
        return carry

    def epil(j, carry):
        drain()
        return carry

    lax.fori_loop(0, K, prol, 0)
    lax.fori_loop(0, NCHD - K, body, 0)
    lax.fori_loop(0, K, epil, 0)
    plsc.subcore_barrier()
    pltpu.sync_copy(deg_sh.at[pl.ds(r0, RPT)], out.at[pl.ds(c * NP + r0, RPT)])


_deg_call = pl.kernel(
    _deg_body,
    out_type=jax.ShapeDtypeStruct((NC * NP, D), jnp.float32),
    mesh=_mesh,
    scratch_types=[
        pltpu.VMEM_SHARED((NP, D), jnp.float32),
        pltpu.VMEM((NCHD, CH), jnp.int32),
        pltpu.VMEM((CH, D), jnp.float32),
        pltpu.SemaphoreType.DMA,
    ],
)


def _scat_body(table, src3, dst3, zerosD, out, acc_sh, sidx, didx, rows, gsem):
    c = lax.axis_index("c")
    s = lax.axis_index("s")
    wid = s * NC + c
    r0 = s * RPT
    pltpu.sync_copy(zerosD.at[pl.ds(r0, RPT)], acc_sh.at[pl.ds(r0, RPT)])
    pltpu.sync_copy(src3.at[wid], sidx)
    pltpu.sync_copy(dst3.at[wid], didx)
    plsc.subcore_barrier()

    nch = jnp.where(c == 0, NCH0, NCH1)

    def body(j, carry):
        pltpu.async_copy(table.at[sidx.at[j]], rows, gsem).wait()
        pltpu.sync_copy(rows, acc_sh.at[didx.at[j]], add=True)
        return carry

    lax.fori_loop(0, nch, body, 0)
    plsc.subcore_barrier()
    pltpu.sync_copy(acc_sh.at[pl.ds(r0, RPT)], out.at[pl.ds(c * NP + r0, RPT)])


_scat_call = pl.kernel(
    _scat_body,
    out_type=jax.ShapeDtypeStruct((NC * NP, D), jnp.float32),
    mesh=_mesh,
    scratch_types=[
        pltpu.VMEM_SHARED((NP, D), jnp.float32),
        pltpu.VMEM((NCHM, CH), jnp.int32),
        pltpu.VMEM((NCHM, CH), jnp.int32),
        pltpu.VMEM((CH, D), jnp.float32),
        pltpu.SemaphoreType.DMA,
    ],
)


def _pad_to(a, n, fill):
    return jnp.concatenate([a, jnp.full((n - a.shape[0],), fill, a.dtype)])


def _core_slab(flat, nch, fill):
    a = flat.reshape(NS, nch, CH)
    if nch < NCHM:
        dum = jnp.full((NS, NCHM - nch, CH), fill, flat.dtype)
        a = jnp.concatenate([a, dum], axis=1)
    return a


def _prep_edges(src, dst):
    dd = _pad_to(dst, EPADD, N).reshape(NW, NCHD, CH)
    s0 = _core_slab(src[:E0], NCH0, 0)
    d0 = _core_slab(dst[:E0], NCH0, N)
    s1 = _core_slab(_pad_to(src[E0:], E1CAP, 0), NCH1, 0)
    d1 = _core_slab(_pad_to(dst[E0:], E1CAP, N), NCH1, N)
    src3 = jnp.stack([s0, s1], axis=1).reshape(NW, NCHM, CH)
    dst3 = jnp.stack([d0, d1], axis=1).reshape(NW, NCHM, CH)
    return dd, src3, dst3


def _pre_body(x_ref, w_ref, d0_ref, d1_ref, hs_ref, dinv_ref):
    x0 = jnp.clip(x_ref[...], -100.0, 100.0)
    deg = d0_ref[...][:, 0:1] + d1_ref[...][:, 0:1] + 1.0
    dinv = lax.rsqrt(deg)
    h = jnp.dot(x0, w_ref[...], preferred_element_type=jnp.float32)
    hs_ref[...] = h * dinv
    dinv_ref[...] = jnp.broadcast_to(dinv, (BR, 16))


_pre_call = pl.pallas_call(
    _pre_body,
    grid=(N // BR,),
    in_specs=[
        pl.BlockSpec((BR, D), lambda i: (i, 0)),
        pl.BlockSpec((D, D), lambda i: (0, 0)),
        pl.BlockSpec((BR, D), lambda i: (i, 0)),
        pl.BlockSpec((BR, D), lambda i: (i, 0)),
    ],
    out_specs=[
        pl.BlockSpec((BR, D), lambda i: (i, 0)),
        pl.BlockSpec((BR, 16), lambda i: (i, 0)),
    ],
    out_shape=[
        jax.ShapeDtypeStruct((N, D), jnp.float32),
        jax.ShapeDtypeStruct((N, 16), jnp.float32),
    ],
)


def _mid_body(p0_ref, p1_ref, hs_ref, dinv_ref, b_ref, w_ref, out_ref):
    dinv = dinv_ref[...][:, 0:1]
    y = dinv * (p0_ref[...] + p1_ref[...] + hs_ref[...]) + b_ref[...]
    y = jnp.maximum(y, 0.0)
    out_ref[...] = jnp.dot(y, w_ref[...], preferred_element_type=jnp.float32) * dinv


_mid_call = pl.pallas_call(
    _mid_body,
    grid=(N // BR,),
    in_specs=[
        pl.BlockSpec((BR, D), lambda i: (i, 0)),
        pl.BlockSpec((BR, D), lambda i: (i, 0)),
        pl.BlockSpec((BR, D), lambda i: (i, 0)),
        pl.BlockSpec((BR, 16), lambda i: (i, 0)),
        pl.BlockSpec((1, D), lambda i: (0, 0)),
        pl.BlockSpec((D, D), lambda i: (0, 0)),
    ],
    out_specs=pl.BlockSpec((BR, D), lambda i: (i, 0)),
    out_shape=jax.ShapeDtypeStruct((N, D), jnp.float32),
)


def _fin_body(q0_ref, q1_ref, hs_ref, dinv_ref, b_ref, x_ref, wh_ref, wx_ref,
              bg_ref, out_ref):
    x0 = jnp.clip(x_ref[...], -100.0, 100.0)
    dinv = dinv_ref[...][:, 0:1]
    h2 = dinv * (q0_ref[...] + q1_ref[...] + hs_ref[...]) + b_ref[...]
    h = jnp.maximum(h2, 0.0) + x0
    g = jax.nn.sigmoid(
        jnp.dot(h, wh_ref[...], preferred_element_type=jnp.float32)
        + jnp.dot(x0, wx_ref[...], preferred_element_type=jnp.float32)
        + bg_ref[...]
    )
    out_ref[...] = g * h + (1.0 - g) * x0


_fin_call = pl.pallas_call(
    _fin_body,
    grid=(N // BR,),
    in_specs=[
        pl.BlockSpec((BR, D), lambda i: (i, 0)),
        pl.BlockSpec((BR, D), lambda i: (i, 0)),
        pl.BlockSpec((BR, D), lambda i: (i, 0)),
        pl.BlockSpec((BR, 16), lambda i: (i, 0)),
        pl.BlockSpec((1, D), lambda i: (0, 0)),
        pl.BlockSpec((BR, D), lambda i: (i, 0)),
        pl.BlockSpec((D, D), lambda i: (0, 0)),
        pl.BlockSpec((D, D), lambda i: (0, 0)),
        pl.BlockSpec((1, D), lambda i: (0, 0)),
    ],
    out_specs=pl.BlockSpec((BR, D), lambda i: (i, 0)),
    out_shape=jax.ShapeDtypeStruct((N, D), jnp.float32),
)


@jax.jit
def kernel(x, edge_index, W1, b1, W2, b2, Wg, bg):
    src = edge_index[0].astype(jnp.int32)
    dst = edge_index[1].astype(jnp.int32)
    dd, src3, dst3 = _prep_edges(src, dst)
    zerosD = jnp.zeros((NP, D), jnp.float32)
    onesD = jnp.ones((CH, D), jnp.float32)

    degp = _deg_call(dd, zerosD, onesD)
    d0, d1 = degp[0:N], degp[NP:NP + N]

    hs1, dinv16 = _pre_call(x, W1, d0, d1)

    acc1 = _scat_call(hs1, src3, dst3, zerosD)
    hs2 = _mid_call(acc1[0:N], acc1[NP:NP + N], hs1, dinv16,
                    b1.reshape(1, D), W2)

    acc2 = _scat_call(hs2, src3, dst3, zerosD)
    out = _fin_call(acc2[0:N], acc2[NP:NP + N], hs2, dinv16,
                    b2.reshape(1, D), x, Wg[:D], Wg[D:], bg.reshape(1, D))
    return out
